# baseline (speedup 1.0000x reference)
import functools

import jax
import jax.numpy as jnp
from jax.experimental import pallas as pl
from jax.experimental.pallas import tpu as pltpu

_IN_DIM = 3
_OUT_DIM = 4
_H1, _H2, _H3 = 64, 128, 256
_HG = 512
_F1, _F2 = 256, 128


def _round_up(a, b):
    return (a + b - 1) // b * b


def _trunk_t(xt, w1_ref, b1_ref, w2_ref, b2_ref, w3_ref, b3_ref):
    h = jnp.maximum(
        jnp.dot(w1_ref[...], xt, preferred_element_type=jnp.float32)
        + b1_ref[:, 0:1], 0.0).astype(jnp.bfloat16)
    h = jnp.maximum(
        jnp.dot(w2_ref[...], h, preferred_element_type=jnp.float32)
        + b2_ref[:, 0:1], 0.0).astype(jnp.bfloat16)
    feat = jnp.maximum(
        jnp.dot(w3_ref[...], h, preferred_element_type=jnp.float32)
        + b3_ref[:, 0:1], 0.0)
    return feat.astype(jnp.bfloat16)


def _max_kernel(xt_ref, w1_ref, b1_ref, w2_ref, b2_ref, w3_ref, b3_ref,
                tmax_ref):
    feat = _trunk_t(xt_ref[...],
                    w1_ref, b1_ref, w2_ref, b2_ref, w3_ref, b3_ref)
    tn = feat.shape[1]
    m = feat[:, 0:128]
    for j in range(1, tn // 128):
        m = jnp.maximum(m, feat[:, 128 * j:128 * (j + 1)])
    tmax_ref[0] = m.astype(jnp.float32)


def _gproj_kernel(tmax_ref, wg_ref, bg_ref, wf1b_ref, bf1_ref, gp_ref):
    pmax = jnp.max(tmax_ref[...], axis=0)
    gmax = jnp.max(pmax, axis=1, keepdims=True)
    gmax = jnp.broadcast_to(gmax, (_H3, 128)).astype(jnp.bfloat16)
    g = jnp.maximum(
        jnp.dot(wg_ref[...], gmax, preferred_element_type=jnp.float32)
        + bg_ref[:, 0:1], 0.0).astype(jnp.bfloat16)
    gp_ref[...] = (
        jnp.dot(wf1b_ref[...], g, preferred_element_type=jnp.float32)
        + bf1_ref[:, 0:1])


def _out_kernel(xt_ref, gp_ref, w1_ref, b1_ref, w2_ref, b2_ref, w3_ref,
                b3_ref, wf1a_ref, wf2_ref, bf2_ref, wo_ref, bo_ref, out_ref):
    feat = _trunk_t(xt_ref[...],
                    w1_ref, b1_ref, w2_ref, b2_ref, w3_ref, b3_ref)
    h = jnp.maximum(
        jnp.dot(wf1a_ref[...], feat, preferred_element_type=jnp.float32)
        + gp_ref[:, 0:1], 0.0).astype(jnp.bfloat16)
    h = jnp.maximum(
        jnp.dot(wf2_ref[...], h, preferred_element_type=jnp.float32)
        + bf2_ref[:, 0:1], 0.0).astype(jnp.bfloat16)
    out_ref[...] = (
        jnp.dot(wo_ref[...], h, preferred_element_type=jnp.float32)
        + bo_ref[:, 0:1]).astype(out_ref.dtype)


@functools.partial(jax.jit, static_argnames=("tile_n",))
def _forward(x, params, tile_n=16384):
    (w1, b1, w2, b2, w3, b3, wg, bg, wf1, bf1, wf2, bf2, wo, bo) = params
    n, in_dim = x.shape

    tn = min(tile_n, _round_up(n, 128))
    n_pad = _round_up(n, tn)
    xt = x.T.astype(jnp.bfloat16)
    if n_pad != n:
        pad = jnp.broadcast_to(xt[:, :1], (in_dim, n_pad - n))
        xt = jnp.concatenate([xt, pad], axis=1)
    num_tiles = n_pad // tn

    bf = jnp.bfloat16

    def tw(w):
        return w.T.astype(bf)

    def tb(b):
        return jnp.broadcast_to(b.T, (b.shape[1], 128))

    w1t, w2t, w3t = tw(w1), tw(w2), tw(w3)
    wgt = tw(wg)
    wf1at, wf1bt = tw(wf1[:_H3]), tw(wf1[_H3:])
    wf2t, wot = tw(wf2), tw(wo)
    b1t, b2t, b3t = tb(b1), tb(b2), tb(b3)
    bgt, bf1t, bf2t, bot = tb(bg), tb(bf1), tb(bf2), tb(bo)

    def const_spec(p):
        return pl.BlockSpec(p.shape, lambda i: (0,) * p.ndim)

    cparams = pltpu.CompilerParams(
        dimension_semantics=("parallel",),
        vmem_limit_bytes=100 * 1024 * 1024,
    )

    s1_params = (w1t, b1t, w2t, b2t, w3t, b3t)
    tmax = pl.pallas_call(
        _max_kernel,
        out_shape=jax.ShapeDtypeStruct((num_tiles, _H3, 128), jnp.float32),
        grid=(num_tiles,),
        in_specs=[pl.BlockSpec((in_dim, tn), lambda i: (0, i))]
                 + [const_spec(p) for p in s1_params],
        out_specs=pl.BlockSpec((1, _H3, 128), lambda i: (i, 0, 0)),
        compiler_params=cparams,
    )(xt, *s1_params)

    g_params = (wgt, bgt, wf1bt, bf1t)
    gp = pl.pallas_call(
        _gproj_kernel,
        out_shape=jax.ShapeDtypeStruct((_H3, 128), jnp.float32),
        grid=(1,),
        in_specs=[pl.BlockSpec(tmax.shape, lambda i: (0, 0, 0))]
                 + [const_spec(p) for p in g_params],
        out_specs=pl.BlockSpec((_H3, 128), lambda i: (0, 0)),
        compiler_params=pltpu.CompilerParams(
            vmem_limit_bytes=100 * 1024 * 1024),
    )(tmax, *g_params)

    s2_params = (w1t, b1t, w2t, b2t, w3t, b3t, wf1at, wf2t, bf2t, wot, bot)
    out_t = pl.pallas_call(
        _out_kernel,
        out_shape=jax.ShapeDtypeStruct((_OUT_DIM, n_pad), jnp.float32),
        grid=(num_tiles,),
        in_specs=[pl.BlockSpec((in_dim, tn), lambda i: (0, i)),
                  pl.BlockSpec((_H3, 128), lambda i: (0, 0))]
                 + [const_spec(p) for p in s2_params],
        out_specs=pl.BlockSpec((_OUT_DIM, tn), lambda i: (0, i)),
        compiler_params=cparams,
    )(xt, gp, *s2_params)

    return out_t[:, :n].T


def kernel(x, w1, b1, w2, b2, w3, b3, wg, bg, wf1, bf1, wf2, bf2, wo, bo):
    params = (w1, b1, w2, b2, w3, b3, wg, bg, wf1, bf1, wf2, bf2, wo, bo)
    return _forward(x, params)


# speedup vs baseline: 1.0028x; 1.0028x over previous
import functools

import numpy as np

import jax
import jax.numpy as jnp
from jax.experimental import pallas as pl
from jax.experimental.pallas import tpu as pltpu
from jax.experimental.shard_map import shard_map
from jax.sharding import Mesh, PartitionSpec as P

_IN_DIM = 3
_OUT_DIM = 4
_H1, _H2, _H3 = 64, 128, 256
_HG = 512
_F1, _F2 = 256, 128


def _round_up(a, b):
    return (a + b - 1) // b * b


def _trunk_t(xt, w1_ref, b1_ref, w2_ref, b2_ref, w3_ref, b3_ref):
    h = jnp.maximum(
        jnp.dot(w1_ref[...], xt, preferred_element_type=jnp.float32)
        + b1_ref[:, 0:1], 0.0).astype(jnp.bfloat16)
    h = jnp.maximum(
        jnp.dot(w2_ref[...], h, preferred_element_type=jnp.float32)
        + b2_ref[:, 0:1], 0.0).astype(jnp.bfloat16)
    feat = jnp.maximum(
        jnp.dot(w3_ref[...], h, preferred_element_type=jnp.float32)
        + b3_ref[:, 0:1], 0.0)
    return feat.astype(jnp.bfloat16)


def _max_kernel(xt_ref, w1_ref, b1_ref, w2_ref, b2_ref, w3_ref, b3_ref,
                tmax_ref):
    feat = _trunk_t(xt_ref[...],
                    w1_ref, b1_ref, w2_ref, b2_ref, w3_ref, b3_ref)
    tn = feat.shape[1]
    m = feat[:, 0:128]
    for j in range(1, tn // 128):
        m = jnp.maximum(m, feat[:, 128 * j:128 * (j + 1)])
    tmax_ref[0] = m.astype(jnp.float32)


def _gproj_kernel(pmax_ref, wg_ref, bg_ref, wf1b_ref, bf1_ref, gp_ref):
    gmax = jnp.max(pmax_ref[...], axis=1, keepdims=True)
    gmax = jnp.broadcast_to(gmax, (_H3, 128)).astype(jnp.bfloat16)
    g = jnp.maximum(
        jnp.dot(wg_ref[...], gmax, preferred_element_type=jnp.float32)
        + bg_ref[:, 0:1], 0.0).astype(jnp.bfloat16)
    gp_ref[...] = (
        jnp.dot(wf1b_ref[...], g, preferred_element_type=jnp.float32)
        + bf1_ref[:, 0:1])


def _out_kernel(xt_ref, gp_ref, w1_ref, b1_ref, w2_ref, b2_ref, w3_ref,
                b3_ref, wf1a_ref, wf2_ref, bf2_ref, wo_ref, bo_ref, out_ref):
    feat = _trunk_t(xt_ref[...],
                    w1_ref, b1_ref, w2_ref, b2_ref, w3_ref, b3_ref)
    h = jnp.maximum(
        jnp.dot(wf1a_ref[...], feat, preferred_element_type=jnp.float32)
        + gp_ref[:, 0:1], 0.0).astype(jnp.bfloat16)
    h = jnp.maximum(
        jnp.dot(wf2_ref[...], h, preferred_element_type=jnp.float32)
        + bf2_ref[:, 0:1], 0.0).astype(jnp.bfloat16)
    out_ref[...] = (
        jnp.dot(wo_ref[...], h, preferred_element_type=jnp.float32)
        + bo_ref[:, 0:1]).astype(out_ref.dtype)


def _local_forward(xt, w1t, b1t, w2t, b2t, w3t, b3t, wgt, bgt, wf1at, wf1bt,
                   bf1t, wf2t, bf2t, wot, bot, *, tn, axis_name):
    n_local = xt.shape[1]
    num_tiles = n_local // tn

    def const_spec(p):
        return pl.BlockSpec(p.shape, lambda i: (0,) * p.ndim)

    cparams = pltpu.CompilerParams(
        dimension_semantics=("arbitrary",),
        vmem_limit_bytes=100 * 1024 * 1024,
    )

    s1_params = (w1t, b1t, w2t, b2t, w3t, b3t)
    tmax = pl.pallas_call(
        _max_kernel,
        out_shape=jax.ShapeDtypeStruct((num_tiles, _H3, 128), jnp.float32),
        grid=(num_tiles,),
        in_specs=[pl.BlockSpec((_IN_DIM, tn), lambda i: (0, i))]
                 + [const_spec(p) for p in s1_params],
        out_specs=pl.BlockSpec((1, _H3, 128), lambda i: (i, 0, 0)),
        compiler_params=cparams,
    )(xt, *s1_params)

    pmax = jnp.max(tmax, axis=0)
    if axis_name is not None:
        pmax = jax.lax.pmax(pmax, axis_name)

    g_params = (wgt, bgt, wf1bt, bf1t)
    gp = pl.pallas_call(
        _gproj_kernel,
        out_shape=jax.ShapeDtypeStruct((_H3, 128), jnp.float32),
        grid=(1,),
        in_specs=[pl.BlockSpec((_H3, 128), lambda i: (0, 0))]
                 + [pl.BlockSpec(p.shape, lambda i: (0, 0)) for p in g_params],
        out_specs=pl.BlockSpec((_H3, 128), lambda i: (0, 0)),
        compiler_params=pltpu.CompilerParams(
            vmem_limit_bytes=100 * 1024 * 1024),
    )(pmax, *g_params)

    s2_params = (w1t, b1t, w2t, b2t, w3t, b3t, wf1at, wf2t, bf2t, wot, bot)
    out_t = pl.pallas_call(
        _out_kernel,
        out_shape=jax.ShapeDtypeStruct((_OUT_DIM, n_local), jnp.float32),
        grid=(num_tiles,),
        in_specs=[pl.BlockSpec((_IN_DIM, tn), lambda i: (0, i)),
                  pl.BlockSpec((_H3, 128), lambda i: (0, 0))]
                 + [const_spec(p) for p in s2_params],
        out_specs=pl.BlockSpec((_OUT_DIM, tn), lambda i: (0, i)),
        compiler_params=cparams,
    )(xt, gp, *s2_params)

    return out_t


def kernel(x, w1, b1, w2, b2, w3, b3, wg, bg, wf1, bf1, wf2, bf2, wo, bo):
    n, in_dim = x.shape
    tile_n = 16384

    devs = jax.devices()
    nd = 2 if len(devs) >= 2 else 1

    tn = min(tile_n, _round_up(n, 128))
    n_pad = _round_up(n, nd * tn)
    xt = x.T.astype(jnp.bfloat16)
    if n_pad != n:
        pad = jnp.broadcast_to(xt[:, :1], (in_dim, n_pad - n))
        xt = jnp.concatenate([xt, pad], axis=1)

    bf = jnp.bfloat16

    def tw(w):
        return w.T.astype(bf)

    def tb(b):
        return jnp.broadcast_to(b.T, (b.shape[1], 128))

    wparams = (tw(w1), tb(b1), tw(w2), tb(b2), tw(w3), tb(b3),
               tw(wg), tb(bg), tw(wf1[:_H3]), tw(wf1[_H3:]), tb(bf1),
               tw(wf2), tb(bf2), tw(wo), tb(bo))

    if nd == 1:
        out_t = _local_forward(xt, *wparams, tn=tn, axis_name=None)
    else:
        mesh = Mesh(np.array(devs[:nd]), ("d",))
        fwd = shard_map(
            functools.partial(_local_forward, tn=tn, axis_name="d"),
            mesh=mesh,
            in_specs=(P(None, "d"),) + (P(None, None),) * len(wparams),
            out_specs=P(None, "d"),
            check_rep=False,
        )
        out_t = fwd(xt, *wparams)

    return out_t[:, :n].T


# speedup vs baseline: 1.0695x; 1.0665x over previous
import functools

import numpy as np

import jax
import jax.numpy as jnp
from jax.experimental import pallas as pl
from jax.experimental.pallas import tpu as pltpu
from jax.experimental.shard_map import shard_map
from jax.sharding import Mesh, PartitionSpec as P

_IN_DIM = 3
_OUT_DIM = 4
_H1, _H2, _H3 = 64, 128, 256
_HG = 512
_F1, _F2 = 256, 128


def _round_up(a, b):
    return (a + b - 1) // b * b


def _trunk_t(xt, w1_ref, b1_ref, w2_ref, b2_ref, w3_ref, b3_ref):
    h = jnp.maximum(
        jnp.dot(w1_ref[...], xt, preferred_element_type=jnp.float32)
        + b1_ref[:, 0:1], 0.0).astype(jnp.bfloat16)
    h = jnp.maximum(
        jnp.dot(w2_ref[...], h, preferred_element_type=jnp.float32)
        + b2_ref[:, 0:1], 0.0).astype(jnp.bfloat16)
    feat = jnp.maximum(
        jnp.dot(w3_ref[...], h, preferred_element_type=jnp.float32)
        + b3_ref[:, 0:1], 0.0)
    return feat.astype(jnp.bfloat16)


def _max_kernel(xt_ref, w1_ref, b1_ref, w2_ref, b2_ref, w3_ref, b3_ref,
                tmax_ref):
    feat = _trunk_t(xt_ref[...],
                    w1_ref, b1_ref, w2_ref, b2_ref, w3_ref, b3_ref)
    tn = feat.shape[1]
    m = feat[:, 0:128]
    for j in range(1, tn // 128):
        m = jnp.maximum(m, feat[:, 128 * j:128 * (j + 1)])
    tmax_ref[0] = m.astype(jnp.float32)


def _gproj_kernel(pmax_ref, wg_ref, bg_ref, wf1b_ref, bf1_ref, gp_ref):
    gmax = jnp.max(pmax_ref[...], axis=1, keepdims=True)
    gmax = jnp.broadcast_to(gmax, (_H3, 128)).astype(jnp.bfloat16)
    g = jnp.maximum(
        jnp.dot(wg_ref[...], gmax, preferred_element_type=jnp.float32)
        + bg_ref[:, 0:1], 0.0).astype(jnp.bfloat16)
    gp_ref[...] = (
        jnp.dot(wf1b_ref[...], g, preferred_element_type=jnp.float32)
        + bf1_ref[:, 0:1])


def _out_kernel(xt_ref, gp_ref, w1_ref, b1_ref, w2_ref, b2_ref, w3_ref,
                b3_ref, wf1a_ref, wf2_ref, bf2_ref, wo_ref, bo_ref, out_ref):
    feat = _trunk_t(xt_ref[...],
                    w1_ref, b1_ref, w2_ref, b2_ref, w3_ref, b3_ref)
    h = jnp.maximum(
        jnp.dot(wf1a_ref[...], feat, preferred_element_type=jnp.float32)
        + gp_ref[:, 0:1], 0.0).astype(jnp.bfloat16)
    h = jnp.maximum(
        jnp.dot(wf2_ref[...], h, preferred_element_type=jnp.float32)
        + bf2_ref[:, 0:1], 0.0).astype(jnp.bfloat16)
    out_ref[...] = (
        jnp.dot(wo_ref[...], h, preferred_element_type=jnp.float32)
        + bo_ref[:, 0:1]).astype(out_ref.dtype)


def _local_forward(x, w1t, b1t, w2t, b2t, w3t, b3t, wgt, bgt, wf1at, wf1bt,
                   bf1t, wf2t, bf2t, wot, bot, *, tn, axis_name):
    xt = x.T.astype(jnp.bfloat16)
    n_local = xt.shape[1]
    num_tiles = n_local // tn

    def const_spec(p):
        return pl.BlockSpec(p.shape, lambda i: (0,) * p.ndim)

    cparams = pltpu.CompilerParams(
        dimension_semantics=("arbitrary",),
        vmem_limit_bytes=100 * 1024 * 1024,
    )

    s1_params = (w1t, b1t, w2t, b2t, w3t, b3t)
    tmax = pl.pallas_call(
        _max_kernel,
        out_shape=jax.ShapeDtypeStruct((num_tiles, _H3, 128), jnp.float32),
        grid=(num_tiles,),
        in_specs=[pl.BlockSpec((_IN_DIM, tn), lambda i: (0, i))]
                 + [const_spec(p) for p in s1_params],
        out_specs=pl.BlockSpec((1, _H3, 128), lambda i: (i, 0, 0)),
        compiler_params=cparams,
    )(xt, *s1_params)

    pmax = jnp.max(tmax, axis=0)
    if axis_name is not None:
        pmax = jax.lax.pmax(pmax, axis_name)

    g_params = (wgt, bgt, wf1bt, bf1t)
    gp = pl.pallas_call(
        _gproj_kernel,
        out_shape=jax.ShapeDtypeStruct((_H3, 128), jnp.float32),
        grid=(1,),
        in_specs=[pl.BlockSpec((_H3, 128), lambda i: (0, 0))]
                 + [pl.BlockSpec(p.shape, lambda i: (0, 0)) for p in g_params],
        out_specs=pl.BlockSpec((_H3, 128), lambda i: (0, 0)),
        compiler_params=pltpu.CompilerParams(
            vmem_limit_bytes=100 * 1024 * 1024),
    )(pmax, *g_params)

    s2_params = (w1t, b1t, w2t, b2t, w3t, b3t, wf1at, wf2t, bf2t, wot, bot)
    out_t = pl.pallas_call(
        _out_kernel,
        out_shape=jax.ShapeDtypeStruct((_OUT_DIM, n_local), jnp.float32),
        grid=(num_tiles,),
        in_specs=[pl.BlockSpec((_IN_DIM, tn), lambda i: (0, i)),
                  pl.BlockSpec((_H3, 128), lambda i: (0, 0))]
                 + [const_spec(p) for p in s2_params],
        out_specs=pl.BlockSpec((_OUT_DIM, tn), lambda i: (0, i)),
        compiler_params=cparams,
    )(xt, gp, *s2_params)

    return out_t.T


def kernel(x, w1, b1, w2, b2, w3, b3, wg, bg, wf1, bf1, wf2, bf2, wo, bo):
    n, in_dim = x.shape
    tile_n = 16384

    devs = jax.devices()
    nd = 2 if len(devs) >= 2 else 1

    tn = min(tile_n, _round_up(n, 128))
    n_pad = _round_up(n, nd * tn)
    if n_pad != n:
        pad = jnp.broadcast_to(x[:1], (n_pad - n, in_dim))
        x = jnp.concatenate([x, pad], axis=0)

    bf = jnp.bfloat16

    def tw(w):
        return w.T.astype(bf)

    def tb(b):
        return jnp.broadcast_to(b.T, (b.shape[1], 128))

    wparams = (tw(w1), tb(b1), tw(w2), tb(b2), tw(w3), tb(b3),
               tw(wg), tb(bg), tw(wf1[:_H3]), tw(wf1[_H3:]), tb(bf1),
               tw(wf2), tb(bf2), tw(wo), tb(bo))

    if nd == 1:
        out = _local_forward(x, *wparams, tn=tn, axis_name=None)
    else:
        mesh = Mesh(np.array(devs[:nd]), ("d",))
        fwd = shard_map(
            functools.partial(_local_forward, tn=tn, axis_name="d"),
            mesh=mesh,
            in_specs=(P("d", None),) + (P(None, None),) * len(wparams),
            out_specs=P("d", None),
            check_rep=False,
        )
        out = fwd(x, *wparams)

    return out[:n]
